# Initial kernel scaffold; baseline (speedup 1.0000x reference)
import jax
import jax.numpy as jnp
from jax.experimental import pallas as pl


def kernel(x, w0, b0, w1, b1, w2, b2):
    raise NotImplementedError("write your pallas kernel here")


# speedup vs baseline: 1.0065x; 1.0065x over previous
import functools

import jax
import jax.numpy as jnp
from jax import lax
from jax.experimental import pallas as pl
from jax.experimental.pallas import tpu as pltpu

_EPS = 1e-5


def _round_up(v, m):
    return (v + m - 1) // m * m


def _apply_act(name, y):
    if name == "leaky":
        return jnp.where(y > 0, y, 0.2 * y)
    if name == "relu":
        return jnp.maximum(y, 0.0)
    return jnp.tanh(jnp.maximum(y, 0.0))


def _block_kernel(p_ref, w_ref, o_ref, y_ref, s1_ref, s2_ref, *,
                  act, inv_p, tile_p):
    ph = pl.program_id(1)
    t = pl.program_id(2)
    row0 = pl.multiple_of(t * tile_p, 8)

    @pl.when(jnp.logical_and(ph == 0, t == 0))
    def _init():
        s1_ref[...] = jnp.zeros_like(s1_ref)
        s2_ref[...] = jnp.zeros_like(s2_ref)

    @pl.when(ph == 0)
    def _matmul_stats():
        y = jnp.dot(p_ref[...], w_ref[...], preferred_element_type=jnp.float32)
        y_ref[pl.ds(row0, tile_p), :] = y
        s1_ref[...] += jnp.sum(y, axis=0, keepdims=True)
        s2_ref[...] += jnp.sum(y * y, axis=0, keepdims=True)

    @pl.when(ph == 1)
    def _normalize_store():
        y = y_ref[pl.ds(row0, tile_p), :]
        mean = s1_ref[...] * inv_p
        var = jnp.maximum(s2_ref[...] * inv_p - mean * mean, 0.0)
        yn = (y - mean) * lax.rsqrt(var + _EPS)
        o_ref[...] = _apply_act(act, yn).astype(o_ref.dtype)


def _conv_in_act(patches, w_mat, act, out_dtype):
    N, P, Kd = patches.shape
    Cout = w_mat.shape[1]
    C_pad = _round_up(Cout, 128)
    itemsize = jnp.dtype(out_dtype).itemsize
    budget = 46 << 20

    tile_p = 128
    for tp in (2048, 1024, 512, 256, 128):
        if tp > _round_up(P, 8):
            continue
        p_pad = _round_up(P, tp)
        need = (p_pad * C_pad * 4
                + 2 * tp * Kd * 2
                + Kd * C_pad * 2
                + 2 * tp * C_pad * itemsize
                + (4 << 20))
        if need <= budget:
            tile_p = tp
            break
    P_pad = _round_up(P, tile_p)
    n_tiles = P_pad // tile_p
    inv_p = 1.0 / float(P)

    if P_pad != P:
        patches = jnp.pad(patches, ((0, 0), (0, P_pad - P), (0, 0)))
    w_pad = jnp.pad(w_mat, ((0, 0), (0, C_pad - Cout))).astype(jnp.bfloat16)

    kern = functools.partial(_block_kernel, act=act, inv_p=inv_p, tile_p=tile_p)
    out = pl.pallas_call(
        kern,
        out_shape=jax.ShapeDtypeStruct((N, P_pad, C_pad), out_dtype),
        grid=(N, 2, n_tiles),
        in_specs=[
            pl.BlockSpec((None, tile_p, Kd),
                         lambda n, ph, t: (n, jnp.where(ph == 0, t, 0), 0)),
            pl.BlockSpec((Kd, C_pad), lambda n, ph, t: (0, 0)),
        ],
        out_specs=pl.BlockSpec((None, tile_p, C_pad),
                               lambda n, ph, t: (n, jnp.where(ph == 1, t, 0), 0)),
        scratch_shapes=[
            pltpu.VMEM((P_pad, C_pad), jnp.float32),
            pltpu.VMEM((1, C_pad), jnp.float32),
            pltpu.VMEM((1, C_pad), jnp.float32),
        ],
        compiler_params=pltpu.CompilerParams(
            dimension_semantics=("parallel", "arbitrary", "arbitrary"),
            vmem_limit_bytes=budget + (4 << 20)),
    )(patches, w_pad)

    return out[:, :P, :Cout]


def _im2col(x_nhwc, k, stride):
    N, H, W, C = x_nhwc.shape
    Ho = (H - k) // stride + 1
    Wo = (W - k) // stride + 1
    cols = [x_nhwc[:, i:i + Ho * stride:stride, j:j + Wo * stride:stride, :]
            for i in range(k) for j in range(k)]
    p = jnp.stack(cols, axis=3)
    return p.reshape(N, Ho * Wo, k * k * C), Ho, Wo


def _w_mat(w_oihw):
    return jnp.transpose(w_oihw, (2, 3, 1, 0)).reshape(-1, w_oihw.shape[0])


def kernel(x, w0, b0, w1, b1, w2, b2):
    del b0, b1, b2
    N = x.shape[0]
    h = jnp.transpose(x, (0, 2, 3, 1)).astype(jnp.bfloat16)

    hp = jnp.pad(h, ((0, 0), (3, 3), (3, 3), (0, 0)), mode="reflect")
    patches, Ho, Wo = _im2col(hp, 7, 1)
    h = _conv_in_act(patches, _w_mat(w0), "leaky", jnp.bfloat16)
    h = h.reshape(N, Ho, Wo, -1)

    hp = jnp.pad(h, ((0, 0), (1, 1), (1, 1), (0, 0)))
    patches, Ho, Wo = _im2col(hp, 4, 2)
    h = _conv_in_act(patches, _w_mat(w1), "relu", jnp.bfloat16)
    h = h.reshape(N, Ho, Wo, -1)

    hp = jnp.pad(h, ((0, 0), (1, 1), (1, 1), (0, 0)))
    patches, Ho, Wo = _im2col(hp, 4, 2)
    h = _conv_in_act(patches, _w_mat(w2), "relu_tanh", jnp.float32)
    h = h.reshape(N, Ho, Wo, -1)

    return jnp.transpose(h, (0, 3, 1, 2))


# speedup vs baseline: 3.1774x; 3.1569x over previous
import functools

import jax
import jax.numpy as jnp
from jax import lax
from jax.experimental import pallas as pl
from jax.experimental.pallas import tpu as pltpu

_EPS = 1e-5


def _round_up(v, m):
    return (v + m - 1) // m * m


def _apply_act(name, y):
    if name == "leaky":
        return jnp.where(y > 0, y, 0.2 * y)
    if name == "relu":
        return jnp.maximum(y, 0.0)
    return jnp.tanh(jnp.maximum(y, 0.0))


def _block_kernel(p_ref, w_ref, o_ref, y_ref, s1_ref, s2_ref, *,
                  act, inv_p, tile_p):
    ph = pl.program_id(1)
    t = pl.program_id(2)
    row0 = pl.multiple_of(t * tile_p, 8)

    @pl.when(jnp.logical_and(ph == 0, t == 0))
    def _init():
        s1_ref[...] = jnp.zeros_like(s1_ref)
        s2_ref[...] = jnp.zeros_like(s2_ref)

    @pl.when(ph == 0)
    def _matmul_stats():
        y = jnp.dot(p_ref[...], w_ref[...], preferred_element_type=jnp.float32)
        y_ref[pl.ds(row0, tile_p), :] = y
        s1_ref[...] += jnp.sum(y, axis=0, keepdims=True)
        s2_ref[...] += jnp.sum(y * y, axis=0, keepdims=True)

    @pl.when(ph == 1)
    def _normalize_store():
        y = y_ref[pl.ds(row0, tile_p), :]
        mean = s1_ref[...] * inv_p
        var = jnp.maximum(s2_ref[...] * inv_p - mean * mean, 0.0)
        yn = (y - mean) * lax.rsqrt(var + _EPS)
        o_ref[...] = _apply_act(act, yn).astype(o_ref.dtype)


def _conv_in_act(patches, w_mat, act, out_dtype):
    N, P, Kd = patches.shape
    Cout = w_mat.shape[1]
    C_pad = _round_up(Cout, 128)
    itemsize = jnp.dtype(out_dtype).itemsize
    budget = 46 << 20

    tile_p = 128
    for tp in (2048, 1024, 512, 256, 128):
        if tp > _round_up(P, 8):
            continue
        p_pad = _round_up(P, tp)
        need = (p_pad * C_pad * 4
                + 2 * tp * Kd * 2
                + Kd * C_pad * 2
                + 2 * tp * C_pad * itemsize
                + (4 << 20))
        if need <= budget:
            tile_p = tp
            break
    P_pad = _round_up(P, tile_p)
    n_tiles = P_pad // tile_p
    inv_p = 1.0 / float(P)

    if P_pad != P:
        patches = jnp.pad(patches, ((0, 0), (0, P_pad - P), (0, 0)))
    w_pad = jnp.pad(w_mat, ((0, 0), (0, C_pad - Cout))).astype(jnp.bfloat16)

    kern = functools.partial(_block_kernel, act=act, inv_p=inv_p, tile_p=tile_p)
    out = pl.pallas_call(
        kern,
        out_shape=jax.ShapeDtypeStruct((N, P_pad, C_pad), out_dtype),
        grid=(N, 2, n_tiles),
        in_specs=[
            pl.BlockSpec((None, tile_p, Kd),
                         lambda n, ph, t: (n, jnp.where(ph == 0, t, 0), 0)),
            pl.BlockSpec((Kd, C_pad), lambda n, ph, t: (0, 0)),
        ],
        out_specs=pl.BlockSpec((None, tile_p, C_pad),
                               lambda n, ph, t: (n, jnp.where(ph == 1, t, 0), 0)),
        scratch_shapes=[
            pltpu.VMEM((P_pad, C_pad), jnp.float32),
            pltpu.VMEM((1, C_pad), jnp.float32),
            pltpu.VMEM((1, C_pad), jnp.float32),
        ],
        compiler_params=pltpu.CompilerParams(
            dimension_semantics=("parallel", "arbitrary", "arbitrary"),
            vmem_limit_bytes=budget + (4 << 20)),
    )(patches, w_pad)

    return out[:, :P, :Cout]


def _ds_kernel(x00_ref, x01_ref, x10_ref, x11_ref, w_ref, o_ref,
               y_ref, s1_ref, s2_ref, *, act, inv_p, th, Wo, C):
    ph = pl.program_id(1)
    t = pl.program_id(2)
    tile_p = th * Wo
    row0 = pl.multiple_of(t * tile_p, 8)
    h0 = t * th
    subs = ((x00_ref, x01_ref), (x10_ref, x11_ref))

    @pl.when(jnp.logical_and(ph == 0, t == 0))
    def _init():
        s1_ref[...] = jnp.zeros_like(s1_ref)
        s2_ref[...] = jnp.zeros_like(s2_ref)

    @pl.when(ph == 0)
    def _matmul_stats():
        acc = jnp.zeros((tile_p, w_ref.shape[1]), jnp.float32)
        for a in (0, 1):
            for b in (0, 1):
                for dh in (0, 1):
                    for dw in (0, 1):
                        kh = 2 * dh + a
                        kw = 2 * dw + b
                        slab = subs[a][b][pl.ds(h0 + dh, th), dw:dw + Wo, :]
                        wblk = w_ref[(kh * 4 + kw) * C:(kh * 4 + kw + 1) * C, :]
                        acc = acc + jnp.dot(slab.reshape(tile_p, C), wblk,
                                            preferred_element_type=jnp.float32)
        y_ref[pl.ds(row0, tile_p), :] = acc
        s1_ref[...] += jnp.sum(acc, axis=0, keepdims=True)
        s2_ref[...] += jnp.sum(acc * acc, axis=0, keepdims=True)

    @pl.when(ph == 1)
    def _normalize_store():
        y = y_ref[pl.ds(row0, tile_p), :]
        mean = s1_ref[...] * inv_p
        var = jnp.maximum(s2_ref[...] * inv_p - mean * mean, 0.0)
        o_ref[...] = _apply_act(act, (y - mean) * lax.rsqrt(var + _EPS)).astype(o_ref.dtype)


def _ds_conv_in_act(h, w_mat, act, out_dtype):
    N, H, W, C = h.shape
    Ho, Wo = H // 2, W // 2
    P = Ho * Wo
    Cout = w_mat.shape[1]
    C_pad = _round_up(Cout, 128)
    hp = jnp.pad(h, ((0, 0), (1, 1), (1, 1), (0, 0)))
    subs = [hp[:, a::2, b::2, :] for a in (0, 1) for b in (0, 1)]
    Hs, Ws = subs[0].shape[1], subs[0].shape[2]
    w_pad = jnp.pad(w_mat, ((0, 0), (0, C_pad - Cout))).astype(jnp.bfloat16)

    th = next(v for v in (8, 4, 2, 1) if Ho % v == 0)
    tile_p = th * Wo
    kern = functools.partial(_ds_kernel, act=act, inv_p=1.0 / float(P),
                            th=th, Wo=Wo, C=C)
    sub_spec = pl.BlockSpec((None, Hs, Ws, C), lambda n, ph, t: (n, 0, 0, 0))
    out = pl.pallas_call(
        kern,
        out_shape=jax.ShapeDtypeStruct((N, P, C_pad), out_dtype),
        grid=(N, 2, Ho // th),
        in_specs=[sub_spec] * 4 + [
            pl.BlockSpec((16 * C, C_pad), lambda n, ph, t: (0, 0))],
        out_specs=pl.BlockSpec((None, tile_p, C_pad),
                               lambda n, ph, t: (n, jnp.where(ph == 1, t, 0), 0)),
        scratch_shapes=[
            pltpu.VMEM((P, C_pad), jnp.float32),
            pltpu.VMEM((1, C_pad), jnp.float32),
            pltpu.VMEM((1, C_pad), jnp.float32),
        ],
        compiler_params=pltpu.CompilerParams(
            dimension_semantics=("parallel", "arbitrary", "arbitrary"),
            vmem_limit_bytes=52 << 20),
    )(*subs, w_pad)
    return out[:, :, :Cout].reshape(N, Ho, Wo, Cout)


def _im2col(x_nhwc, k, stride):
    N, H, W, C = x_nhwc.shape
    Ho = (H - k) // stride + 1
    Wo = (W - k) // stride + 1
    cols = [x_nhwc[:, i:i + Ho * stride:stride, j:j + Wo * stride:stride, :]
            for i in range(k) for j in range(k)]
    p = jnp.stack(cols, axis=3)
    return p.reshape(N, Ho * Wo, k * k * C), Ho, Wo


def _w_mat(w_oihw):
    return jnp.transpose(w_oihw, (2, 3, 1, 0)).reshape(-1, w_oihw.shape[0])


def kernel(x, w0, b0, w1, b1, w2, b2):
    del b0, b1, b2
    N = x.shape[0]
    h = jnp.transpose(x, (0, 2, 3, 1)).astype(jnp.bfloat16)

    hp = jnp.pad(h, ((0, 0), (3, 3), (3, 3), (0, 0)), mode="reflect")
    patches, Ho, Wo = _im2col(hp, 7, 1)
    h = _conv_in_act(patches, _w_mat(w0), "leaky", jnp.bfloat16)
    h = h.reshape(N, Ho, Wo, -1)

    h = _ds_conv_in_act(h, _w_mat(w1), "relu", jnp.bfloat16)
    h = _ds_conv_in_act(h, _w_mat(w2), "relu_tanh", jnp.float32)

    return jnp.transpose(h, (0, 3, 1, 2))
